# Initial kernel scaffold; baseline (speedup 1.0000x reference)
import jax
import jax.numpy as jnp
from jax.experimental import pallas as pl


def kernel(x, Wgate, Wg, Wu, Wd):
    raise NotImplementedError("write your pallas kernel here")


# speedup vs baseline: 1.1474x; 1.1474x over previous
import functools

import jax
import jax.numpy as jnp
from jax.experimental import pallas as pl
from jax.experimental.pallas import tpu as pltpu

B, S, D = 1, 2048, 768
E, K, F = 8, 2, 1024
T = B * S
BM = 256


def _router_body(x_ref, wg_ref, wdense_ref, eidx_ref, wtop_ref):
    x = x_ref[...]
    wg = wg_ref[...]
    logits = jax.lax.dot_general(
        x, wg, (((1,), (1,)), ((), ())),
        preferred_element_type=jnp.float32)
    iota = jax.lax.broadcasted_iota(jnp.int32, (BM, E), 1)
    m1 = jnp.max(logits, axis=1, keepdims=True)
    big = jnp.int32(2**30)
    i1 = jnp.min(jnp.where(logits == m1, iota, big), axis=1, keepdims=True)
    masked = jnp.where(iota == i1, -jnp.inf, logits)
    m2 = jnp.max(masked, axis=1, keepdims=True)
    i2 = jnp.min(jnp.where(masked == m2, iota, big), axis=1, keepdims=True)
    e2 = jnp.exp(m2 - m1)
    w1 = 1.0 / (1.0 + e2)
    w2 = e2 / (1.0 + e2)
    wdense = jnp.where(iota == i1, w1, 0.0) + jnp.where(iota == i2, w2, 0.0)
    wdense_ref[...] = wdense
    eidx_ref[...] = jnp.concatenate([i1, i2], axis=1)
    wtop_ref[...] = jnp.concatenate([w1, w2], axis=1)


def _router(xf, Wgate):
    return pl.pallas_call(
        _router_body,
        grid=(T // BM,),
        in_specs=[
            pl.BlockSpec((BM, D), lambda t: (t, 0)),
            pl.BlockSpec((E, D), lambda t: (0, 0)),
        ],
        out_specs=[
            pl.BlockSpec((BM, E), lambda t: (t, 0)),
            pl.BlockSpec((BM, K), lambda t: (t, 0)),
            pl.BlockSpec((BM, K), lambda t: (t, 0)),
        ],
        out_shape=[
            jax.ShapeDtypeStruct((T, E), jnp.float32),
            jax.ShapeDtypeStruct((T, K), jnp.int32),
            jax.ShapeDtypeStruct((T, K), jnp.float32),
        ],
    )(xf, Wgate)


def _ffn_dense_body(x_ref, wgp_ref, wup_ref, wdp_ref, wdense_ref, y_ref):
    e = pl.program_id(1)
    x = x_ref[...]
    wgp = wgp_ref[0]
    wup = wup_ref[0]
    wdp = wdp_ref[0]
    xg = jax.lax.dot_general(x, wgp, (((1,), (1,)), ((), ())),
                             preferred_element_type=jnp.float32)
    xu = jax.lax.dot_general(x, wup, (((1,), (1,)), ((), ())),
                             preferred_element_type=jnp.float32)
    h = (xg * jax.nn.sigmoid(xg)) * xu
    out = jax.lax.dot_general(h, wdp, (((1,), (1,)), ((), ())),
                              preferred_element_type=jnp.float32)
    iota = jax.lax.broadcasted_iota(jnp.int32, (BM, E), 1)
    wcol = jnp.sum(wdense_ref[...] * (iota == e), axis=1, keepdims=True)
    contrib = out * wcol

    @pl.when(e == 0)
    def _():
        y_ref[...] = contrib

    @pl.when(e != 0)
    def _():
        y_ref[...] = y_ref[...] + contrib


def _ffn_dense(xf, Wg, Wu, Wd, wdense):
    return pl.pallas_call(
        _ffn_dense_body,
        grid=(T // BM, E),
        in_specs=[
            pl.BlockSpec((BM, D), lambda t, e: (t, 0)),
            pl.BlockSpec((1, F, D), lambda t, e: (e, 0, 0)),
            pl.BlockSpec((1, F, D), lambda t, e: (e, 0, 0)),
            pl.BlockSpec((1, D, F), lambda t, e: (e, 0, 0)),
            pl.BlockSpec((BM, E), lambda t, e: (t, 0)),
        ],
        out_specs=pl.BlockSpec((BM, D), lambda t, e: (t, 0)),
        out_shape=jax.ShapeDtypeStruct((T, D), jnp.float32),
    )(xf, Wg, Wu, Wd, wdense)


def kernel(x, Wgate, Wg, Wu, Wd):
    xf = x.reshape(T, D)
    wdense, eidx, wtop = _router(xf, Wgate)
    y = _ffn_dense(xf, Wg, Wu, Wd, wdense)
    return y.reshape(B, S, D)


# speedup vs baseline: 1.7120x; 1.4921x over previous
import functools

import jax
import jax.numpy as jnp
from jax import lax
from jax.experimental import pallas as pl
from jax.experimental.pallas import tpu as pltpu
from jax.experimental.pallas import tpu_sc as plsc

B, S, D = 1, 2048, 768
E, K, F = 8, 2, 1024
T = B * S
A = T * K
BM = 256
BM_SHIFT = 8
PA = A + E * BM
G = PA // BM
BR = 256

NC, NS, L = 2, 16, 16
NW = NC * NS
APW = A // NW
TPW = T // NW


def _router_body(x_ref, wg_ref, eidx_ref, wtop_ref, rank_ref, counts_ref,
                 poff_ref, cnt_scr):
    t = pl.program_id(0)

    @pl.when(t == 0)
    def _():
        cnt_scr[...] = jnp.zeros_like(cnt_scr)

    x = x_ref[...]
    wg = wg_ref[...]
    logits = lax.dot_general(x, wg, (((1,), (1,)), ((), ())),
                             preferred_element_type=jnp.float32)
    iota = lax.broadcasted_iota(jnp.int32, (BR, E), 1)
    big = jnp.int32(2**30)
    m1 = jnp.max(logits, axis=1, keepdims=True)
    i1 = jnp.min(jnp.where(logits == m1, iota, big), axis=1, keepdims=True)
    masked = jnp.where(iota == i1, -jnp.inf, logits)
    m2 = jnp.max(masked, axis=1, keepdims=True)
    i2 = jnp.min(jnp.where(masked == m2, iota, big), axis=1, keepdims=True)
    e2 = jnp.exp(m2 - m1)
    w1 = 1.0 / (1.0 + e2)
    w2 = e2 / (1.0 + e2)

    oh0 = (iota == i1).astype(jnp.float32)
    oh1 = (iota == i2).astype(jnp.float32)
    ohp = oh0 + oh1
    ri = lax.broadcasted_iota(jnp.int32, (BR, BR), 0)
    ci = lax.broadcasted_iota(jnp.int32, (BR, BR), 1)
    tri = (ci < ri).astype(jnp.float32)
    cum = lax.dot_general(tri, ohp, (((1,), (0,)), ((), ())),
                          preferred_element_type=jnp.float32)
    base = cnt_scr[...] + cum
    rank0 = jnp.sum(oh0 * base, axis=1, keepdims=True)
    rank1 = jnp.sum(oh1 * base, axis=1, keepdims=True)
    cnt_new = cnt_scr[...] + jnp.sum(ohp, axis=0, keepdims=True)
    cnt_scr[...] = cnt_new

    eidx_ref[...] = jnp.concatenate([i1, i2], axis=1)
    wtop_ref[...] = jnp.concatenate([w1, w2], axis=1)
    rank_ref[...] = jnp.concatenate([rank0, rank1], axis=1).astype(jnp.int32)
    counts_ref[...] = cnt_new.astype(jnp.int32)
    pcnt = jnp.floor((cnt_new + (BM - 1)) * (1.0 / BM)) * BM
    ei = lax.broadcasted_iota(jnp.int32, (E, E), 0)
    ej = lax.broadcasted_iota(jnp.int32, (E, E), 1)
    stri = (ei < ej).astype(jnp.float32)
    poff = lax.dot_general(pcnt, stri, (((1,), (0,)), ((), ())),
                           preferred_element_type=jnp.float32)
    pad = jnp.zeros((1, 16 - E), jnp.int32)
    poff_ref[...] = jnp.concatenate([poff.astype(jnp.int32), pad], axis=1)


def _router(xf, Wgate):
    return pl.pallas_call(
        _router_body,
        grid=(T // BR,),
        in_specs=[
            pl.BlockSpec((BR, D), lambda t: (t, 0)),
            pl.BlockSpec((E, D), lambda t: (0, 0)),
        ],
        out_specs=[
            pl.BlockSpec((BR, K), lambda t: (t, 0)),
            pl.BlockSpec((BR, K), lambda t: (t, 0)),
            pl.BlockSpec((BR, K), lambda t: (t, 0)),
            pl.BlockSpec((1, E), lambda t: (0, 0)),
            pl.BlockSpec((1, 16), lambda t: (0, 0)),
        ],
        out_shape=[
            jax.ShapeDtypeStruct((T, K), jnp.int32),
            jax.ShapeDtypeStruct((T, K), jnp.float32),
            jax.ShapeDtypeStruct((T, K), jnp.int32),
            jax.ShapeDtypeStruct((1, E), jnp.int32),
            jax.ShapeDtypeStruct((1, 16), jnp.int32),
        ],
        scratch_shapes=[pltpu.VMEM((1, E), jnp.float32)],
    )(xf, Wgate)


@functools.cache
def _get_sc_dispatch():
    @functools.partial(
        pl.kernel,
        out_type=jax.ShapeDtypeStruct((PA, D), jnp.float32),
        mesh=plsc.VectorSubcoreMesh(core_axis_name="c", subcore_axis_name="s",
                                    num_cores=NC, num_subcores=NS),
        compiler_params=pltpu.CompilerParams(needs_layout_passes=False),
        scratch_types=[
            pltpu.VMEM((16,), jnp.int32),
            pltpu.VMEM((APW,), jnp.int32),
            pltpu.VMEM((APW,), jnp.int32),
            pltpu.VMEM((APW,), jnp.int32),
            pltpu.VMEM((APW,), jnp.int32),
            pltpu.VMEM((APW, D), jnp.float32),
            pltpu.SemaphoreType.DMA,
            pltpu.SemaphoreType.DMA,
        ],
    )
    def _sc_dispatch(xf_hbm, eflat_hbm, rflat_hbm, poff_hbm, xs_hbm,
                     poff_vm, e_vm, r_vm, pos_vm, tok_vm, rows_vm,
                     sem1, sem2):
        wid = lax.axis_index("s") * NC + lax.axis_index("c")
        abase = wid * APW
        pltpu.sync_copy(poff_hbm, poff_vm)
        pltpu.sync_copy(eflat_hbm.at[pl.ds(abase, APW)], e_vm)
        pltpu.sync_copy(rflat_hbm.at[pl.ds(abase, APW)], r_vm)
        for i in range(APW // L):
            sl = pl.ds(i * L, L)
            e_v = e_vm[sl]
            r_v = r_vm[sl]
            pos_vm[sl] = plsc.load_gather(poff_vm, [e_v]) + r_v
            j_v = abase + i * L + lax.iota(jnp.int32, L)
            tok_vm[sl] = j_v >> 1
        pltpu.async_copy(xf_hbm.at[tok_vm], rows_vm, sem1).wait()
        pltpu.async_copy(rows_vm, xs_hbm.at[pos_vm], sem2).wait()

    return _sc_dispatch


def _ffn_body(be_ref, xs_ref, wg_ref, wu_ref, wd_ref, out_ref):
    x = xs_ref[...]
    xg = lax.dot_general(x, wg_ref[0], (((1,), (1,)), ((), ())),
                         preferred_element_type=jnp.float32)
    xu = lax.dot_general(x, wu_ref[0], (((1,), (1,)), ((), ())),
                         preferred_element_type=jnp.float32)
    h = (xg * jax.nn.sigmoid(xg)) * xu
    out_ref[...] = lax.dot_general(h, wd_ref[0], (((1,), (1,)), ((), ())),
                                   preferred_element_type=jnp.float32)


def _ffn_grouped(be, xs, Wg, Wu, Wd):
    grid_spec = pltpu.PrefetchScalarGridSpec(
        num_scalar_prefetch=1,
        grid=(G,),
        in_specs=[
            pl.BlockSpec((BM, D), lambda b, be_ref: (b, 0)),
            pl.BlockSpec((1, F, D), lambda b, be_ref: (be_ref[b], 0, 0)),
            pl.BlockSpec((1, F, D), lambda b, be_ref: (be_ref[b], 0, 0)),
            pl.BlockSpec((1, D, F), lambda b, be_ref: (be_ref[b], 0, 0)),
        ],
        out_specs=pl.BlockSpec((BM, D), lambda b, be_ref: (b, 0)),
    )
    return pl.pallas_call(
        _ffn_body,
        grid_spec=grid_spec,
        out_shape=jax.ShapeDtypeStruct((PA, D), jnp.float32),
    )(be, xs, Wg, Wu, Wd)


@functools.cache
def _get_sc_gather_back():
    @functools.partial(
        pl.kernel,
        out_type=jax.ShapeDtypeStruct((K, T, D), jnp.float32),
        mesh=plsc.VectorSubcoreMesh(core_axis_name="c", subcore_axis_name="s",
                                    num_cores=NC, num_subcores=NS),
        compiler_params=pltpu.CompilerParams(needs_layout_passes=False),
        scratch_types=[
            pltpu.VMEM((16,), jnp.int32),
            pltpu.VMEM((APW,), jnp.int32),
            pltpu.VMEM((APW,), jnp.int32),
            pltpu.VMEM((APW,), jnp.int32),
            pltpu.VMEM((APW, D), jnp.float32),
            pltpu.SemaphoreType.DMA,
        ],
    )
    def _sc_gather_back(outp_hbm, eflat_hbm, rflat_hbm, poff_hbm, gp_hbm,
                        poff_vm, e_vm, r_vm, pos_vm, rows_vm, sem1):
        wid = lax.axis_index("s") * NC + lax.axis_index("c")
        abase = wid * APW
        tbase = wid * TPW
        pltpu.sync_copy(poff_hbm, poff_vm)
        pltpu.sync_copy(eflat_hbm.at[pl.ds(abase, APW)], e_vm)
        pltpu.sync_copy(rflat_hbm.at[pl.ds(abase, APW)], r_vm)
        for i in range(APW // L):
            k = i // (TPW // L)
            l_v = (i % (TPW // L)) * L + lax.iota(jnp.int32, L)
            a_v = 2 * l_v + k
            e_v = plsc.load_gather(e_vm, [a_v])
            r_v = plsc.load_gather(r_vm, [a_v])
            pos_vm[pl.ds(i * L, L)] = plsc.load_gather(poff_vm, [e_v]) + r_v
        pltpu.async_copy(outp_hbm.at[pos_vm], rows_vm, sem1).wait()
        pltpu.sync_copy(rows_vm.at[pl.ds(0, TPW)],
                        gp_hbm.at[0, pl.ds(tbase, TPW)])
        pltpu.sync_copy(rows_vm.at[pl.ds(TPW, TPW)],
                        gp_hbm.at[1, pl.ds(tbase, TPW)])

    return _sc_gather_back


def _combine_body(gp_ref, w_ref, y_ref):
    g = gp_ref[...]
    w = w_ref[...]
    y_ref[...] = g[0] * w[:, 0:1] + g[1] * w[:, 1:2]


def _combine(gp, wtop):
    return pl.pallas_call(
        _combine_body,
        grid=(T // BM,),
        in_specs=[
            pl.BlockSpec((K, BM, D), lambda t: (0, t, 0)),
            pl.BlockSpec((BM, K), lambda t: (t, 0)),
        ],
        out_specs=pl.BlockSpec((BM, D), lambda t: (t, 0)),
        out_shape=jax.ShapeDtypeStruct((T, D), jnp.float32),
    )(gp, wtop)


def kernel(x, Wgate, Wg, Wu, Wd):
    xf = x.reshape(T, D)
    eidx, wtop, rank, counts, poff16 = _router(xf, Wgate)

    eflat = eidx.reshape(A)
    rflat = rank.reshape(A)
    poff_flat = poff16.reshape(16)

    pcnt = ((counts[0] + (BM - 1)) // BM) * BM
    cums = jnp.cumsum(pcnt)
    starts = jnp.arange(G, dtype=jnp.int32) * BM
    be = jnp.minimum(jnp.sum(starts[:, None] >= cums[None, :], axis=1),
                     E - 1).astype(jnp.int32)

    xs = _get_sc_dispatch()(xf, eflat, rflat, poff_flat)
    outp = _ffn_grouped(be, xs, Wg, Wu, Wd)
    gp = _get_sc_gather_back()(outp, eflat, rflat, poff_flat)
    y = _combine(gp, wtop)
    return y.reshape(B, S, D)
